# baseline (device time: 42966 ns/iter reference)
import jax
import jax.numpy as jnp
from jax import lax
from jax.experimental import pallas as pl
from jax.experimental.pallas import tpu as pltpu

CHUNKS_PER_B = 2
AMAX = 4.0


def kernel(O, Wo):
    B, S, Hl, D = O.shape
    K = Hl * D
    N = Wo.shape[1]
    S_half = S // 2
    n_chunks = B * CHUNKS_PER_B
    rows = S_half // CHUNKS_PER_B

    O3 = O.reshape(B, S, K)
    Wo_bf = Wo.astype(jnp.bfloat16)

    def body(o_hbm, w_ref, out_hbm, o_vmem, out_vmem, send_ref, recv_ref,
             load_sems, store_sems, send_sems, recv_sems):
        my_x = lax.axis_index("x")
        my_y = lax.axis_index("y")
        my_z = lax.axis_index("z")
        other_z = 1 - my_z
        partner = (my_x, my_y, other_z)

        def o_load(slot, half_z, i):
            b, c = divmod(i, CHUNKS_PER_B)
            start = half_z * S_half + c * rows
            return pltpu.make_async_copy(
                o_hbm.at[b, pl.ds(start, rows), :], o_vmem.at[slot],
                load_sems.at[slot],
            )

        for i in range(n_chunks):
            o_load(i, other_z, i).start()
        for i in range(n_chunks):
            o_load(n_chunks + i, my_z, i).start()

        barrier = pltpu.get_barrier_semaphore()
        pl.semaphore_signal(barrier, inc=1, device_id=partner,
                            device_id_type=pl.DeviceIdType.MESH)
        pl.semaphore_wait(barrier, 1)

        def chunk_rdma(i):
            return pltpu.make_async_remote_copy(
                src_ref=send_ref.at[i], dst_ref=recv_ref.at[i],
                send_sem=send_sems.at[i], recv_sem=recv_sems.at[i],
                device_id=partner, device_id_type=pl.DeviceIdType.MESH,
            )

        for i in range(n_chunks):
            o_load(i, other_z, i).wait()
            blk = o_vmem[i].astype(jnp.bfloat16)
            p = jnp.dot(blk, w_ref[...], preferred_element_type=jnp.float32)
            send_ref[i] = jnp.clip(
                jnp.round(p * (127.0 / AMAX)), -127.0, 127.0
            ).astype(jnp.int8)
            chunk_rdma(i).start()

        for i in range(n_chunks):
            b, c = divmod(i, CHUNKS_PER_B)
            o_load(n_chunks + i, my_z, i).wait()
            blk = o_vmem[n_chunks + i].astype(jnp.bfloat16)
            out_vmem[b, pl.ds(c * rows, rows), :] = jnp.dot(
                blk, w_ref[...], preferred_element_type=jnp.float32
            )

        for i in range(n_chunks):
            chunk_rdma(i).wait_recv()
            b, c = divmod(i, CHUNKS_PER_B)
            sl = pl.ds(c * rows, rows)
            out_vmem[b, sl, :] += recv_ref[i].astype(jnp.float32) * (AMAX / 127.0)
            pltpu.make_async_copy(
                out_vmem.at[b, sl, :], out_hbm.at[b, sl, :], store_sems.at[i]
            ).start()

        for i in range(n_chunks):
            b, c = divmod(i, CHUNKS_PER_B)
            sl = pl.ds(c * rows, rows)
            pltpu.make_async_copy(
                out_vmem.at[b, sl, :], out_hbm.at[b, sl, :], store_sems.at[i]
            ).wait()
        for i in range(n_chunks):
            chunk_rdma(i).wait_send()

    return pl.pallas_call(
        body,
        out_shape=jax.ShapeDtypeStruct((B, S_half, N), jnp.float32),
        in_specs=[
            pl.BlockSpec(memory_space=pl.ANY),
            pl.BlockSpec(memory_space=pltpu.VMEM),
        ],
        out_specs=pl.BlockSpec(memory_space=pl.ANY),
        scratch_shapes=[
            pltpu.VMEM((2 * n_chunks, rows, K), jnp.float32),
            pltpu.VMEM((B, S_half, N), jnp.float32),
            pltpu.VMEM((n_chunks, rows, N), jnp.int8),
            pltpu.VMEM((n_chunks, rows, N), jnp.int8),
            pltpu.SemaphoreType.DMA((2 * n_chunks,)),
            pltpu.SemaphoreType.DMA((n_chunks,)),
            pltpu.SemaphoreType.DMA((n_chunks,)),
            pltpu.SemaphoreType.DMA((n_chunks,)),
        ],
        compiler_params=pltpu.CompilerParams(collective_id=0),
    )(O3, Wo_bf)


# device time: 41006 ns/iter; 1.0478x vs baseline; 1.0478x over previous
import jax
import jax.numpy as jnp
from jax import lax
from jax.experimental import pallas as pl
from jax.experimental.pallas import tpu as pltpu

CHUNKS_PER_B = 4
AMAX = 4.0


def kernel(O, Wo):
    B, S, Hl, D = O.shape
    K = Hl * D
    N = Wo.shape[1]
    S_half = S // 2
    n_chunks = B * CHUNKS_PER_B
    rows = S_half // CHUNKS_PER_B

    O3 = O.reshape(B, S, K)

    def body(o_ref, w_ref, out_ref, w_bf_ref, send_ref, recv_ref,
             send_sems, recv_sems):
        my_x = lax.axis_index("x")
        my_y = lax.axis_index("y")
        my_z = lax.axis_index("z")
        other_z = 1 - my_z
        partner = (my_x, my_y, other_z)

        barrier = pltpu.get_barrier_semaphore()
        pl.semaphore_signal(barrier, inc=1, device_id=partner,
                            device_id_type=pl.DeviceIdType.MESH)
        pl.semaphore_wait(barrier, 1)

        w_bf_ref[...] = w_ref[...].astype(jnp.bfloat16)

        def chunk_rdma(i):
            return pltpu.make_async_remote_copy(
                src_ref=send_ref.at[i], dst_ref=recv_ref.at[i],
                send_sem=send_sems.at[i], recv_sem=recv_sems.at[i],
                device_id=partner, device_id_type=pl.DeviceIdType.MESH,
            )

        for i in range(n_chunks):
            b, c = divmod(i, CHUNKS_PER_B)
            start = other_z * S_half + c * rows
            blk = o_ref[b, pl.ds(start, rows), :].astype(jnp.bfloat16)
            p = jnp.dot(blk, w_bf_ref[...], preferred_element_type=jnp.float32)
            send_ref[i] = jnp.clip(
                jnp.round(p * (127.0 / AMAX)), -127.0, 127.0
            ).astype(jnp.int8)
            chunk_rdma(i).start()

        for b in range(B):
            blk = o_ref[b, pl.ds(my_z * S_half, S_half), :].astype(jnp.bfloat16)
            out_ref[b] = jnp.dot(
                blk, w_bf_ref[...], preferred_element_type=jnp.float32
            )

        for i in range(n_chunks):
            chunk_rdma(i).wait_recv()
            b, c = divmod(i, CHUNKS_PER_B)
            out_ref[b, pl.ds(c * rows, rows), :] += (
                recv_ref[i].astype(jnp.float32) * (AMAX / 127.0)
            )

        for i in range(n_chunks):
            chunk_rdma(i).wait_send()

    return pl.pallas_call(
        body,
        out_shape=jax.ShapeDtypeStruct((B, S_half, N), jnp.float32),
        in_specs=[
            pl.BlockSpec(memory_space=pltpu.VMEM),
            pl.BlockSpec(memory_space=pltpu.VMEM),
        ],
        out_specs=pl.BlockSpec(memory_space=pltpu.VMEM),
        scratch_shapes=[
            pltpu.VMEM((K, N), jnp.bfloat16),
            pltpu.VMEM((n_chunks, rows, N), jnp.int8),
            pltpu.VMEM((n_chunks, rows, N), jnp.int8),
            pltpu.SemaphoreType.DMA((n_chunks,)),
            pltpu.SemaphoreType.DMA((n_chunks,)),
        ],
        compiler_params=pltpu.CompilerParams(collective_id=0),
    )(O3, Wo)


# device time: 40759 ns/iter; 1.0541x vs baseline; 1.0061x over previous
import jax
import jax.numpy as jnp
from jax import lax
from jax.experimental import pallas as pl
from jax.experimental.pallas import tpu as pltpu

CHUNKS_PER_B = 4
AMAX = 4.0


def kernel(O, Wo):
    B, S, Hl, D = O.shape
    K = Hl * D
    N = Wo.shape[1]
    S_half = S // 2
    n_chunks = B * CHUNKS_PER_B
    rows = S_half // CHUNKS_PER_B

    O3 = O.reshape(B, S, K)

    def body(o_ref, w_ref, out_hbm, w_bf_ref, out_ref, send_ref, recv_ref,
             send_sems, recv_sems, store_sems):
        my_x = lax.axis_index("x")
        my_y = lax.axis_index("y")
        my_z = lax.axis_index("z")
        other_z = 1 - my_z
        partner = (my_x, my_y, other_z)

        barrier = pltpu.get_barrier_semaphore()
        pl.semaphore_signal(barrier, inc=1, device_id=partner,
                            device_id_type=pl.DeviceIdType.MESH)
        pl.semaphore_wait(barrier, 1)

        w_bf_ref[...] = w_ref[...].astype(jnp.bfloat16)

        def chunk_rdma(i):
            return pltpu.make_async_remote_copy(
                src_ref=send_ref.at[i], dst_ref=recv_ref.at[i],
                send_sem=send_sems.at[i], recv_sem=recv_sems.at[i],
                device_id=partner, device_id_type=pl.DeviceIdType.MESH,
            )

        for i in range(n_chunks):
            b, c = divmod(i, CHUNKS_PER_B)
            start = other_z * S_half + c * rows
            blk = o_ref[b, pl.ds(start, rows), :].astype(jnp.bfloat16)
            p = jnp.dot(blk, w_bf_ref[...], preferred_element_type=jnp.float32)
            send_ref[i] = jnp.clip(
                jnp.round(p * (127.0 / AMAX)), -127.0, 127.0
            ).astype(jnp.int8)
            chunk_rdma(i).start()

        for b in range(B):
            blk = o_ref[b, pl.ds(my_z * S_half, S_half), :].astype(jnp.bfloat16)
            out_ref[b] = jnp.dot(
                blk, w_bf_ref[...], preferred_element_type=jnp.float32
            )

        def store_copy(i):
            b, c = divmod(i, CHUNKS_PER_B)
            sl = pl.ds(c * rows, rows)
            return pltpu.make_async_copy(
                out_ref.at[b, sl, :], out_hbm.at[b, sl, :], store_sems.at[i]
            )

        for i in range(n_chunks):
            chunk_rdma(i).wait_recv()
            b, c = divmod(i, CHUNKS_PER_B)
            out_ref[b, pl.ds(c * rows, rows), :] += (
                recv_ref[i].astype(jnp.float32) * (AMAX / 127.0)
            )
            store_copy(i).start()

        for i in range(n_chunks):
            store_copy(i).wait()
        for i in range(n_chunks):
            chunk_rdma(i).wait_send()

    return pl.pallas_call(
        body,
        out_shape=jax.ShapeDtypeStruct((B, S_half, N), jnp.float32),
        in_specs=[
            pl.BlockSpec(memory_space=pltpu.VMEM),
            pl.BlockSpec(memory_space=pltpu.VMEM),
        ],
        out_specs=pl.BlockSpec(memory_space=pl.ANY),
        scratch_shapes=[
            pltpu.VMEM((K, N), jnp.bfloat16),
            pltpu.VMEM((B, S_half, N), jnp.float32),
            pltpu.VMEM((n_chunks, rows, N), jnp.int8),
            pltpu.VMEM((n_chunks, rows, N), jnp.int8),
            pltpu.SemaphoreType.DMA((n_chunks,)),
            pltpu.SemaphoreType.DMA((n_chunks,)),
            pltpu.SemaphoreType.DMA((n_chunks,)),
        ],
        compiler_params=pltpu.CompilerParams(collective_id=0),
    )(O3, Wo)
